# Initial kernel scaffold; baseline (speedup 1.0000x reference)
import jax
import jax.numpy as jnp
from jax.experimental import pallas as pl


def kernel(pts, W_enc, W_samp):
    raise NotImplementedError("write your pallas kernel here")


# speedup vs baseline: 3.6438x; 3.6438x over previous
import jax
import jax.numpy as jnp
from jax.experimental import pallas as pl

_B = 8
_N = 32768
_ROWS = _N // 128
_INT_MIN = -(2**31)


def _count(mask):
    return jnp.sum(mask.astype(jnp.int32))


def _search32(pred, key, r):
    def vbody(i, t):
        cand = t | jnp.left_shift(jnp.int32(1), 31 - i)
        cnt = _count(pred & (key < (cand ^ _INT_MIN)))
        return jnp.where(cnt <= r, cand, t)

    t = jax.lax.fori_loop(0, 32, vbody, jnp.int32(0))
    return t ^ _INT_MIN


def _find(pred, keys, idx, r):
    vstars = []
    for key in keys:
        v = _search32(pred, key, r)
        vstars.append(v)
        r = r - _count(pred & (key < v))
        pred = pred & (key == v)

    def ibody(i, t2):
        cand = t2 | jnp.left_shift(jnp.int32(1), 14 - i)
        cnt = _count(pred & (idx < cand))
        return jnp.where(cnt <= r, cand, t2)

    istar = jax.lax.fori_loop(0, 15, ibody, jnp.int32(0))
    return vstars, istar


def _member(pred, keys, idx, vstars, istar):
    eq = pred
    less = jnp.full(idx.shape, False)
    for key, v in zip(keys, vstars):
        less = less | (eq & (key < v))
        eq = eq & (key == v)
    return less | (eq & (idx <= istar))


def _sortable(x):
    bits = jax.lax.bitcast_convert_type(x, jnp.int32)
    return bits ^ (jnp.right_shift(bits, 31) & jnp.int32(0x7FFFFFFF))


def _body(s0_ref, s1_ref, s2_ref, cho_ref, w_ref, out_ref):
    s0 = s0_ref[0]
    s1 = s1_ref[0]
    s2 = s2_ref[0]
    k0 = _sortable(s0)
    k1 = _sortable(s1)
    k2 = _sortable(s2)
    idx = (
        jax.lax.broadcasted_iota(jnp.int32, (_ROWS, 128), 0) * 128
        + jax.lax.broadcasted_iota(jnp.int32, (_ROWS, 128), 1)
    )
    full = jnp.full((_ROWS, 128), True)

    masks0 = []
    for d0 in (k0, ~k0):
        v, i = _find(full, [d0], idx, jnp.int32(20479))
        masks0.append(_member(full, [d0], idx, v, i))

    masks1 = []
    dirs1 = []
    for b0, m0 in enumerate(masks0):
        d0 = k0 if b0 == 0 else ~k0
        for d1 in (k1, ~k1):
            v, i = _find(m0, [d1, d0], idx, jnp.int32(12799))
            masks1.append(_member(m0, [d1, d0], idx, v, i))
            dirs1.append((d1, d0))

    for j in range(8):
        b0 = j & 1
        b1 = (j >> 1) & 1
        b2 = (j >> 2) & 1
        pred = masks1[2 * b0 + b1]
        d1, d0 = dirs1[2 * b0 + b1]
        d2 = k2 if b2 == 0 else ~k2
        r = cho_ref[0, 0, j]
        _, istar = _find(pred, [d2, d1, d0], idx, r)
        hot = idx == istar
        sel0 = jnp.sum(jnp.where(hot, s0, 0.0))
        sel1 = jnp.sum(jnp.where(hot, s1, 0.0))
        sel2 = jnp.sum(jnp.where(hot, s2, 0.0))
        out_ref[0, j : j + 1, :] = (
            sel0 * w_ref[0:1, :] + sel1 * w_ref[1:2, :] + sel2 * w_ref[2:3, :]
        )


@jax.jit
def kernel(pts, W_enc, W_samp):
    enc = pts @ W_enc
    enc_t = jnp.transpose(enc, (2, 0, 1)).reshape(3, _B, _ROWS, 128)
    cho = jax.random.randint(
        jax.random.key(42), (_B, 8, 1), 0, 8000
    ).astype(jnp.int32)
    cho3 = jnp.transpose(cho, (0, 2, 1))
    w_pad = jnp.zeros((8, 128), jnp.float32).at[:3].set(W_samp)

    grid = (_B,)
    chan = pl.BlockSpec((1, _ROWS, 128), lambda b: (b, 0, 0))
    out = pl.pallas_call(
        _body,
        grid=grid,
        in_specs=[
            chan,
            chan,
            chan,
            pl.BlockSpec((1, 1, 8), lambda b: (b, 0, 0)),
            pl.BlockSpec((8, 128), lambda b: (0, 0)),
        ],
        out_specs=pl.BlockSpec((1, 8, 128), lambda b: (b, 0, 0)),
        out_shape=jax.ShapeDtypeStruct((_B, 8, 128), jnp.float32),
    )(enc_t[0], enc_t[1], enc_t[2], cho3, w_pad)
    return out


# speedup vs baseline: 11.7794x; 3.2328x over previous
import functools

import jax
import jax.numpy as jnp
from jax import lax
from jax.experimental import pallas as pl
from jax.experimental.pallas import tpu as pltpu
from jax.experimental.pallas import tpu_sc as plsc

_B = 8
_N = 32768
_ITERS = _N // 16
_INT_MIN = -(2**31)
_INT_MAX = 2**31 - 1

_R = 0
_TIE = 1
_PFX = 2
_V0 = 3
_IST = 7
_M0 = 8


def _signed(mask32):
    return mask32 - 2**32 if mask32 >= 2**31 else mask32


def _extract(vec, lane, iota):
    return jnp.sum(jnp.where(iota == lane, vec, 0))


def _emit_find(st, hist, binsbuf, iota, laneoff, words, predfn, rank):
    st[_R] = rank
    st[_TIE] = jnp.int32(2**30)
    st[_PFX] = jnp.int32(0)
    for w in range(4):
        st[_V0 + w] = jnp.int32(0)
        st[_M0 + w] = jnp.int32(0)
    st[_IST] = jnp.int32(_INT_MAX)

    ones = jnp.full((16,), 1, jnp.int32)

    def word_uk(w, i, iv):
        loadfn, is_idx = words[w]
        ek = loadfn(i, iv)
        return ek if is_idx else ek ^ _INT_MIN

    def chain_eq(i, iv, upto):
        m = jnp.full((16,), True)
        for w in range(upto):
            vw = jnp.full((16,), st[_V0 + w])
            m = m & (word_uk(w, i, iv) == vw)
        return m

    for w, (loadfn, is_idx) in enumerate(words):
        nbytes = 2 if is_idx else 4
        for bp in range(nbytes):
            shift = 8 * (nbytes - 1 - bp)
            himask = _signed((~((1 << (shift + 8)) - 1)) & 0xFFFFFFFF)

            @pl.when(st[_TIE] > 1)
            def _(w=w, bp=bp, shift=shift, himask=himask, loadfn=loadfn,
                  is_idx=is_idx, nbytes=nbytes):
                def zbody(i, c):
                    hist[pl.ds(i * 16, 16)] = jnp.zeros((16,), jnp.int32)
                    return c

                lax.fori_loop(0, 256, zbody, 0)

                pref = jnp.full((16,), st[_PFX])
                rscal = st[_R]

                def sbody(i, c):
                    iv = iota + i * 16
                    ek = loadfn(i, iv)
                    uk = ek if is_idx else ek ^ _INT_MIN
                    m = predfn(i, iv) & chain_eq(i, iv, w)
                    if bp > 0:
                        m = m & (((uk ^ pref) & himask) == 0)
                    d = lax.shift_right_logical(uk, shift) & 255
                    plsc.addupdate_scatter(hist, [d + laneoff], ones, mask=m)
                    return c

                lax.fori_loop(0, _ITERS, sbody, 0)

                def cbody(cidx, cs):
                    def lbody(l, acc):
                        return acc + hist[pl.ds(l * 256 + cidx * 16, 16)]

                    acc = lax.fori_loop(0, 16, lbody, jnp.zeros((16,), jnp.int32))
                    binsbuf[pl.ds(cidx * 16, 16)] = acc
                    return jnp.where(iota == cidx, jnp.sum(acc), cs)

                cs = lax.fori_loop(0, 16, cbody, jnp.zeros((16,), jnp.int32))
                cums = plsc.cumsum(cs)
                rvec = jnp.full((16,), rscal)
                mA = cums <= rvec
                cstar = jnp.max(jnp.where(mA, iota + 1, 0))
                belowA = jnp.sum(jnp.where(mA, cs, 0))
                binvec = binsbuf[pl.ds(cstar * 16, 16)]
                c2 = plsc.cumsum(binvec)
                mB = (c2 + belowA) <= rvec
                lstar = jnp.max(jnp.where(mB, iota + 1, 0))
                belowB = belowA + jnp.sum(jnp.where(mB, binvec, 0))
                digit = cstar * 16 + lstar
                cnt_at = _extract(binvec, lstar, iota)

                newpfx = st[_PFX] | (digit << shift)
                st[_R] = rscal - belowB
                st[_TIE] = cnt_at
                st[_V0 + w] = newpfx
                st[_M0 + w] = st[_M0 + w] | _signed((255 << shift) & 0xFFFFFFFF)
                if bp == nbytes - 1:
                    if is_idx:
                        st[_IST] = newpfx
                    st[_PFX] = jnp.int32(0)
                else:
                    st[_PFX] = newpfx

    @pl.when(st[_IST] == _INT_MAX)
    def _():
        def ebody(i, acc):
            iv = iota + i * 16
            m = predfn(i, iv)
            for w in range(len(words)):
                vv = jnp.full((16,), st[_V0 + w])
                mk = jnp.full((16,), st[_M0 + w])
                m = m & ((word_uk(w, i, iv) & mk) == vv)
            return jnp.maximum(acc, jnp.max(jnp.where(m, iv, -1)))

        st[_IST] = lax.fori_loop(0, _ITERS, ebody, jnp.int32(-1))


def _sc_body(enc_hbm, cho_hbm, out_hbm, kflat, hist, binsbuf, chobuf,
             outvec, st):
    cax = lax.axis_index("c")
    sax = lax.axis_index("s")
    b_local = sax // 4
    q = sax % 4
    b = cax * 4 + b_local
    b0 = q & 1
    b1 = q >> 1

    iota = lax.broadcasted_iota(jnp.int32, (16,), 0)
    laneoff = iota * 256

    for c in range(3):
        pltpu.sync_copy(enc_hbm.at[c, b], kflat.at[pl.ds(c * _N, _N)])
    pltpu.sync_copy(cho_hbm.at[b], chobuf)

    def conv(i, c):
        x = kflat[pl.ds(i * 16, 16)]
        kflat[pl.ds(i * 16, 16)] = x ^ (lax.shift_right_arithmetic(x, 31) & _INT_MAX)
        return c

    lax.fori_loop(0, 3 * _ITERS, conv, 0)

    dm0 = jnp.full((16,), -b0)
    dm1 = jnp.full((16,), -b1)

    def keyload(ch, dmvec):
        def f(i, iv):
            return kflat[pl.ds(ch * _N + i * 16, 16)] ^ dmvec

        return f

    def idxload(i, iv):
        return iv

    true_pred = lambda i, iv: jnp.full((16,), True)

    _emit_find(st, hist, binsbuf, iota, laneoff,
               [(keyload(0, dm0), False), (idxload, True)],
               true_pred, jnp.int32(20479))
    i0b = jnp.clip(jnp.full((16,), st[_IST]), 0, _N - 1)

    k0d = keyload(0, dm0)
    k1d = keyload(1, dm1)

    v0b = plsc.load_gather(kflat, [i0b]) ^ dm0

    def pred1(i, iv):
        e0 = k0d(i, iv)
        return (e0 < v0b) | ((e0 == v0b) & (iv <= i0b))

    _emit_find(st, hist, binsbuf, iota, laneoff,
               [(k1d, False), (k0d, False), (idxload, True)],
               pred1, jnp.int32(12799))
    i1b = jnp.clip(jnp.full((16,), st[_IST]), 0, _N - 1)
    v1b = plsc.load_gather(kflat, [i1b + _N]) ^ dm1
    v0t = plsc.load_gather(kflat, [i1b]) ^ dm0

    def pred2(i, iv):
        e1 = k1d(i, iv)
        e0 = k0d(i, iv)
        m0 = (e0 < v0b) | ((e0 == v0b) & (iv <= i0b))
        inner = (e0 < v0t) | ((e0 == v0t) & (iv <= i1b))
        return m0 & ((e1 < v1b) | ((e1 == v1b) & inner))

    chov = chobuf[...]

    for b2 in range(2):
        j = 4 * b2 + 2 * b1 + b0
        cho_j = _extract(chov, j, iota)
        dm2 = jnp.full((16,), -b2)
        _emit_find(st, hist, binsbuf, iota, laneoff,
                   [(keyload(2, dm2), False), (k1d, False), (k0d, False),
                    (idxload, True)],
                   pred2, cho_j)
        pos = jnp.clip(jnp.full((16,), st[_IST]), 0, _N - 1)
        coord = jnp.zeros((16,), jnp.float32)
        for ch in range(3):
            kv = plsc.load_gather(kflat, [pos + ch * _N])
            raw = kv ^ (lax.shift_right_arithmetic(kv, 31) & _INT_MAX)
            fv = plsc.bitcast(raw, jnp.float32)
            coord = jnp.where(iota == ch, fv, coord)
        outvec[...] = coord
        pltpu.sync_copy(outvec, out_hbm.at[b * 8 + j])


def _tc_project(sel_ref, w_ref, out_ref):
    sel = sel_ref[...]
    out_ref[...] = (
        sel[:, 0:1] * w_ref[0:1, :]
        + sel[:, 1:2] * w_ref[1:2, :]
        + sel[:, 2:3] * w_ref[2:3, :]
    )


@jax.jit
def kernel(pts, W_enc, W_samp):
    enc = pts @ W_enc
    enc_i = lax.bitcast_convert_type(jnp.transpose(enc, (2, 0, 1)), jnp.int32)
    cho = jax.random.randint(jax.random.key(42), (_B, 8, 1), 0, 8000)
    cho_pad = jnp.zeros((_B, 16), jnp.int32).at[:, :8].set(
        cho[:, :, 0].astype(jnp.int32)
    )

    mesh = plsc.VectorSubcoreMesh(core_axis_name="c", subcore_axis_name="s")
    sel = pl.kernel(
        _sc_body,
        out_type=jax.ShapeDtypeStruct((_B * 8, 16), jnp.float32),
        mesh=mesh,
        compiler_params=pltpu.CompilerParams(needs_layout_passes=False),
        scratch_types=[
            pltpu.VMEM((3 * _N,), jnp.int32),
            pltpu.VMEM((4096,), jnp.int32),
            pltpu.VMEM((256,), jnp.int32),
            pltpu.VMEM((16,), jnp.int32),
            pltpu.VMEM((16,), jnp.float32),
            pltpu.SMEM((16,), jnp.int32),
        ],
    )(enc_i, cho_pad)

    w_pad = jnp.zeros((8, 128), jnp.float32).at[:3].set(W_samp)
    feat = pl.pallas_call(
        _tc_project,
        out_shape=jax.ShapeDtypeStruct((_B * 8, 128), jnp.float32),
    )(sel, w_pad)
    return feat.reshape(_B, 8, 128)
